# Initial kernel scaffold; baseline (speedup 1.0000x reference)
import jax
import jax.numpy as jnp
from jax.experimental import pallas as pl


def kernel(x, edge_index, edge_weight, W1, W2):
    raise NotImplementedError("write your pallas kernel here")


# speedup vs baseline: 8.2304x; 8.2304x over previous
import functools

import jax
import jax.numpy as jnp
from jax import lax
from jax.experimental import pallas as pl
from jax.experimental.pallas import tpu as pltpu
from jax.experimental.pallas import tpu_sc as plsc

N = 10000
E = 320000
EP = 327680
D_IN = 128
D_HID = 128
D_OUT = 64

P = 10240
NC = 2
NS = 16
NW = NC * NS
EPT = EP // NW
C = 64
NCH = EPT // C
RPT = P // NS
DEGW = 16

_mesh = plsc.VectorSubcoreMesh(core_axis_name="c", subcore_axis_name="s",
                               num_cores=NC, num_subcores=NS)


def _zero_shared(zbuf, acc_sh, d, sid):
  def zrow(r, _):
    for k in range(d // 16):
      zbuf[r, pl.ds(k * 16, 16)] = jnp.zeros((16,), jnp.float32)
    return _
  lax.fori_loop(0, C, zrow, None)
  def zcopy(j, _):
    pltpu.sync_copy(zbuf, acc_sh.at[pl.ds(sid * RPT + j * C, C)])
    return _
  lax.fori_loop(0, RPT // C, zcopy, None)


@functools.partial(
    pl.kernel,
    out_type=jax.ShapeDtypeStruct((NC, P, DEGW), jnp.float32),
    mesh=_mesh,
    scratch_types=[
        pltpu.VMEM((NCH, C), jnp.int32),
        pltpu.VMEM((EPT,), jnp.float32),
        pltpu.VMEM((C, DEGW), jnp.float32),
        pltpu.VMEM_SHARED((P, DEGW), jnp.float32),
    ],
)
def _sc_deg(row2d, ew, degp, rowbuf, ewbuf, ewrep, acc_sh):
  cid = lax.axis_index("c")
  sid = lax.axis_index("s")
  wid = cid * NS + sid
  _zero_shared(ewrep, acc_sh, DEGW, sid)
  pltpu.sync_copy(row2d.at[pl.ds(wid * NCH, NCH)], rowbuf)
  pltpu.sync_copy(ew.at[pl.ds(wid * EPT, EPT)], ewbuf)
  plsc.subcore_barrier()

  def chunk(i, _):
    def rep(g, _):
      vew = ewbuf[pl.ds(i * C + g * 16, 16)]
      for e in range(16):
        ewrep[g * 16 + e, :] = jnp.full((DEGW,), vew[e], jnp.float32)
      return _
    lax.fori_loop(0, C // 16, rep, None)
    pltpu.sync_copy(ewrep, acc_sh.at[rowbuf.at[i]], add=True)
    return _
  lax.fori_loop(0, NCH, chunk, None)

  plsc.subcore_barrier()
  pltpu.sync_copy(acc_sh.at[pl.ds(sid * RPT, RPT)],
                  degp.at[cid, pl.ds(sid * RPT, RPT)])


def _make_sc_msg(d):
  @functools.partial(
      pl.kernel,
      out_type=jax.ShapeDtypeStruct((NC, P, d), jnp.float32),
      mesh=_mesh,
      scratch_types=[
          pltpu.VMEM((EPT // 2,), jnp.int32),
          pltpu.VMEM((NCH // 2, C), jnp.int32),
          pltpu.VMEM((EPT // 2,), jnp.float32),
          pltpu.VMEM((C, d), jnp.float32),
          pltpu.VMEM((C, d), jnp.float32),
          pltpu.VMEM_SHARED((P, d), jnp.float32),
          pltpu.SemaphoreType.DMA,
          pltpu.SemaphoreType.DMA,
      ],
  )
  def sc_msg(y, col, row2d, ew, accp,
             colbuf, rowbuf, ewbuf, rows0, rows1, acc_sh, sem0, sem1):
    cid = lax.axis_index("c")
    sid = lax.axis_index("s")
    wid = cid * NS + sid
    _zero_shared(rows0, acc_sh, d, sid)
    plsc.subcore_barrier()

    bufs = (rows0, rows1)
    sems = (sem0, sem1)
    EH = EPT // 2
    NCHH = NCH // 2

    for h in range(2):
      pltpu.sync_copy(col.at[pl.ds(wid * EPT + h * EH, EH)], colbuf)
      pltpu.sync_copy(row2d.at[pl.ds(wid * NCH + h * NCHH, NCHH)], rowbuf)
      pltpu.sync_copy(ew.at[pl.ds(wid * EPT + h * EH, EH)], ewbuf)

      def gather(i, b):
        return pltpu.async_copy(y.at[colbuf.at[pl.ds(i * C, C)]], bufs[b],
                                sems[b])

      def scale_scatter(i, b):
        rows = bufs[b]
        def group(g, _):
          vew = ewbuf[pl.ds(i * C + g * 16, 16)]
          for e in range(16):
            s = vew[e]
            for k in range(d // 16):
              sl = pl.ds(k * 16, 16)
              rows[g * 16 + e, sl] = rows[g * 16 + e, sl] * s
          return _
        lax.fori_loop(0, C // 16, group, None)
        pltpu.sync_copy(rows, acc_sh.at[rowbuf.at[i]], add=True)

      gather(0, 0).wait()
      def pair(p, _):
        i0 = p * 2
        c1 = gather(i0 + 1, 1)
        scale_scatter(i0, 0)
        c1.wait()
        @pl.when(i0 + 2 < NCHH)
        def _pref():
          gather(i0 + 2, 0).wait()
        scale_scatter(i0 + 1, 1)
        return _
      lax.fori_loop(0, NCHH // 2, pair, None)

    plsc.subcore_barrier()
    pltpu.sync_copy(acc_sh.at[pl.ds(sid * RPT, RPT)],
                    accp.at[cid, pl.ds(sid * RPT, RPT)])

  return sc_msg


_sc_msg_hid = _make_sc_msg(D_HID)

BR = 1024


def _tc_a_body(degp_ref, x_ref, w_ref, y_ref, dinv_ref):
  deg = jnp.sum(degp_ref[...], axis=(0, 2)) * (1.0 / DEGW) + 1.0
  dinv = lax.rsqrt(deg)
  xw = jnp.dot(x_ref[...], w_ref[...], preferred_element_type=jnp.float32)
  y_ref[...] = xw * dinv[:, None]
  dinv_ref[...] = jnp.broadcast_to(dinv[:, None], (BR, 128))


def _tc_b_body(accp_ref, y1_ref, dinv_ref, w_ref, y2_ref):
  acc = accp_ref[0] + accp_ref[1] + y1_ref[...]
  h = jnp.maximum(acc * dinv_ref[...], 0.0)
  xw2 = jnp.dot(h, w_ref[...], preferred_element_type=jnp.float32)
  y2_ref[...] = xw2 * dinv_ref[...]


BRC = 1000


def _tc_c_body(accp_ref, y2_ref, dinv_ref, out_ref):
  zf = (accp_ref[0] + accp_ref[1] + y2_ref[...]) * dinv_ref[...]
  z = zf[:, :D_OUT]
  m = jnp.max(z, axis=1, keepdims=True)
  s = z - m
  lse = jnp.log(jnp.sum(jnp.exp(s), axis=1, keepdims=True))
  out_ref[...] = s - lse


def kernel(x, edge_index, edge_weight, W1, W2):
  ew = jnp.pad(edge_weight.reshape(E).astype(jnp.float32), (0, EP - E))
  row = jnp.pad(edge_index[0].astype(jnp.int32), (0, EP - E))
  col = jnp.pad(edge_index[1].astype(jnp.int32), (0, EP - E))
  row2d = row.reshape(EP // C, C)
  x_pad = jnp.pad(x, ((0, P - N), (0, 0)))
  w2p = jnp.pad(W2, ((0, 0), (0, D_HID - D_OUT)))

  degp = _sc_deg(row2d, ew)

  y1, dinvb = pl.pallas_call(
      _tc_a_body,
      grid=(P // BR,),
      in_specs=[
          pl.BlockSpec((NC, BR, DEGW), lambda i: (0, i, 0)),
          pl.BlockSpec((BR, D_IN), lambda i: (i, 0)),
          pl.BlockSpec((D_IN, D_HID), lambda i: (0, 0)),
      ],
      out_specs=[
          pl.BlockSpec((BR, D_HID), lambda i: (i, 0)),
          pl.BlockSpec((BR, 128), lambda i: (i, 0)),
      ],
      out_shape=[
          jax.ShapeDtypeStruct((P, D_HID), jnp.float32),
          jax.ShapeDtypeStruct((P, 128), jnp.float32),
      ],
  )(degp, x_pad, W1)

  accp1 = _sc_msg_hid(y1, col, row2d, ew)

  y2 = pl.pallas_call(
      _tc_b_body,
      grid=(P // BR,),
      in_specs=[
          pl.BlockSpec((NC, BR, D_HID), lambda i: (0, i, 0)),
          pl.BlockSpec((BR, D_HID), lambda i: (i, 0)),
          pl.BlockSpec((BR, 128), lambda i: (i, 0)),
          pl.BlockSpec((D_HID, D_HID), lambda i: (0, 0)),
      ],
      out_specs=pl.BlockSpec((BR, D_HID), lambda i: (i, 0)),
      out_shape=jax.ShapeDtypeStruct((P, D_HID), jnp.float32),
  )(accp1, y1, dinvb, w2p)

  accp2 = _sc_msg_hid(y2, col, row2d, ew)

  out = pl.pallas_call(
      _tc_c_body,
      grid=(N // BRC,),
      in_specs=[
          pl.BlockSpec((NC, BRC, D_HID), lambda i: (0, i, 0)),
          pl.BlockSpec((BRC, D_HID), lambda i: (i, 0)),
          pl.BlockSpec((BRC, 128), lambda i: (i, 0)),
      ],
      out_specs=pl.BlockSpec((BRC, D_OUT), lambda i: (i, 0)),
      out_shape=jax.ShapeDtypeStruct((N, D_OUT), jnp.float32),
  )(accp2, y2, dinvb)

  return out
